# Initial kernel scaffold; baseline (speedup 1.0000x reference)
import jax
import jax.numpy as jnp
from jax.experimental import pallas as pl


def kernel(x, Win, bin_, Wout, bout):
    raise NotImplementedError("write your pallas kernel here")


# speedup vs baseline: 2.7967x; 2.7967x over previous
import numpy as np
import jax
import jax.numpy as jnp
from jax.experimental import pallas as pl
from jax.experimental.pallas import tpu as pltpu

_G = 2
_R = 2
_DIM = 1024
_DPG = _DIM // _G
_CDIM = 4
_NB = 8
_T = 2048
_TT = 512
_NT = _T // _TT
_TOKENS = _NB * _T
_EPS = np.float32(1e-5)

_HALF_L = np.float32(
    np.float32(np.float32(5.0) - np.float32(1.0))
    * np.float32(1.0 + 1e-3)
    / np.float32(2.0)
)


def _gfsq_tc(x_ref, win_ref, bin_ref, wout_ref, bout_ref,
             feat_ref, ind_ref, perp_ref, hist_ref):
    b = pl.program_id(0)
    t = pl.program_id(1)

    @pl.when((b == 0) & (t == 0))
    def _init():
        hist_ref[...] = jnp.zeros_like(hist_ref)

    xb = x_ref[0]
    sub = jax.lax.broadcasted_iota(jnp.int32, (32, _TT), 0)
    ind_rows = []
    for g in range(_G):
        xg = xb[g * _DPG:(g + 1) * _DPG, :]
        wg = win_ref[g * _CDIM:(g + 1) * _CDIM, :]
        z = jax.lax.dot_general(
            wg, xg, (((1,), (0,)), ((), ())),
            preferred_element_type=jnp.float32)
        z = z + bin_ref[g * _CDIM:(g + 1) * _CDIM, 0:1]
        qout = jnp.zeros_like(z)
        r = z
        for i in range(_R):
            inv_scale = np.float32(4.0 ** i)
            scale = np.float32(4.0 ** (-i))
            q = jnp.round(jnp.tanh(r * inv_scale) * _HALF_L)
            codes = q * np.float32(0.5)
            zhat = q + np.float32(2.0)
            lo = zhat[0:1, :] + np.float32(5.0) * zhat[1:2, :]
            hi = zhat[2:3, :] + np.float32(5.0) * zhat[3:4, :]
            quant = codes * scale
            r = r - quant
            qout = qout + quant
            ind_rows.append(lo + np.float32(25.0) * hi)
            a_oh = (sub == lo.astype(jnp.int32)).astype(jnp.bfloat16)
            b_oh = (sub == hi.astype(jnp.int32)).astype(jnp.bfloat16)
            cnt = jax.lax.dot_general(
                a_oh, b_oh, (((1,), (1,)), ((), ())),
                preferred_element_type=jnp.float32)
            k = g * _R + i
            hist_ref[k] = hist_ref[k] + cnt
        wo = wout_ref[g * _DPG:(g + 1) * _DPG, :]
        f = jax.lax.dot_general(
            wo, qout, (((1,), (0,)), ((), ())),
            preferred_element_type=jnp.float32)
        f = f + bout_ref[g * _DPG:(g + 1) * _DPG, 0:1]
        feat_ref[0, g * _DPG:(g + 1) * _DPG, :] = f
    ind_ref[0] = jnp.concatenate(
        [row.astype(jnp.int32) for row in ind_rows], axis=0)

    @pl.when((b == _NB - 1) & (t == _NT - 1))
    def _finish():
        lane = jax.lax.broadcasted_iota(jnp.int32, (1, 128), 1)
        acc = jnp.zeros((1, 128), jnp.float32)
        for c in range(_G * _R):
            e = hist_ref[c] * np.float32(1.0 / _TOKENS)
            s = jnp.sum(e)
            e = e / (s + _EPS)
            ent = e * jnp.log(e + _EPS)
            sc = jnp.sum(ent)
            acc = jnp.where(lane == c, -sc, acc)
        perp_ref[...] = jnp.exp(acc)


def kernel(x, Win, bin_, Wout, bout):
    winr = Win.reshape(_G * _CDIM, _DPG)
    binr = bin_.reshape(_G * _CDIM, 1)
    woutr = Wout.reshape(_G * _DPG, _CDIM)
    boutr = bout.reshape(_G * _DPG, 1)
    feat, ind, perp = pl.pallas_call(
        _gfsq_tc,
        grid=(_NB, _NT),
        in_specs=[
            pl.BlockSpec((1, _DIM, _TT), lambda b, t: (b, 0, t)),
            pl.BlockSpec((_G * _CDIM, _DPG), lambda b, t: (0, 0)),
            pl.BlockSpec((_G * _CDIM, 1), lambda b, t: (0, 0)),
            pl.BlockSpec((_G * _DPG, _CDIM), lambda b, t: (0, 0)),
            pl.BlockSpec((_G * _DPG, 1), lambda b, t: (0, 0)),
        ],
        out_specs=[
            pl.BlockSpec((1, _DIM, _TT), lambda b, t: (b, 0, t)),
            pl.BlockSpec((1, _G * _R, _TT), lambda b, t: (b, 0, t)),
            pl.BlockSpec((1, 128), lambda b, t: (0, 0)),
        ],
        out_shape=[
            jax.ShapeDtypeStruct((_NB, _DIM, _T), jnp.float32),
            jax.ShapeDtypeStruct((_NB, _G * _R, _T), jnp.int32),
            jax.ShapeDtypeStruct((1, 128), jnp.float32),
        ],
        scratch_shapes=[pltpu.VMEM((_G * _R, 32, 32), jnp.float32)],
    )(x, winr, binr, woutr, boutr)
    p = perp[0, :_G * _R]
    return (jnp.zeros_like(p), feat, p, ind)
